# baseline (speedup 1.0000x reference)
import functools

import numpy as np
import jax
import jax.numpy as jnp
from jax import lax
from jax.experimental import pallas as pl
from jax.experimental.pallas import tpu as pltpu
from jax.experimental.pallas import tpu_sc as plsc

T = 32768
L = 4
RES = [16, 24, 36, 54]
ROWS = [(r + 1) ** 3 if (r + 1) ** 3 <= T else T for r in RES]
P2I = int(np.int32(np.uint32(2654435761)))
P3I = int(np.int32(np.uint32(805459861)))
M16 = -65536
NW_PAD = [-(-2 * r // 512) * 512 for r in ROWS]

NC, NS = 2, 16
NW = NC * NS
WPG = NW // L
CP = 2048

BN = 16384


def _sc_encode_body(x_hbm, y_hbm, z_hbm, tbl_hbm, *rest):
    outs = rest[:16]
    xv_r, yv_r, zv_r, tbl_v, f0v, f1v, f2v, f3v = rest[16:]
    npts_total = x_hbm.shape[0]
    npts = npts_total // WPG
    wid = lax.axis_index("s") * NC + lax.axis_index("c")
    g = wid // WPG
    lw = wid % WPG

    for lvl in range(L):
        @pl.when(g == lvl)
        def _(lvl=lvl):
            res = RES[lvl]
            r1 = res + 1
            dense = r1 ** 3 <= T
            nwords = NW_PAD[lvl]
            pltpu.sync_copy(tbl_hbm.at[pl.ds(lvl * 2 * T, nwords)],
                            tbl_v.at[pl.ds(0, nwords)])
            base = lw * npts

            def chunk_body(c, _):
                cbase = base + c * CP
                sl_hbm = pl.ds(cbase, CP)
                pltpu.sync_copy(x_hbm.at[sl_hbm], xv_r)
                pltpu.sync_copy(y_hbm.at[sl_hbm], yv_r)
                pltpu.sync_copy(z_hbm.at[sl_hbm], zv_r)

                @plsc.parallel_loop(0, CP // 16, 1, unroll=4)
                def vreg_body(j):
                    sl = pl.ds(j * 16, 16)
                    x = xv_r[sl]
                    y = yv_r[sl]
                    z = zv_r[sl]
                    fres = float(res)
                    px = x * fres
                    py = y * fres
                    pz = z * fres
                    cx = px.astype(jnp.int32)
                    cy = py.astype(jnp.int32)
                    cz = pz.astype(jnp.int32)
                    wx = px - cx.astype(jnp.float32)
                    wy = py - cy.astype(jnp.float32)
                    wz = pz - cz.astype(jnp.float32)
                    ux = 1.0 - wx
                    uy = 1.0 - wy
                    uz = 1.0 - wz
                    axy = (ux * uy, wx * uy, ux * wy, wx * wy)
                    if dense:
                        tx = (cx * 2, cx * 2 + 2)
                        ty = (cy * (2 * r1), (cy + 1) * (2 * r1))
                        tz = (cz * (2 * r1 * r1), (cz + 1) * (2 * r1 * r1))
                    else:
                        tx = (cx * 2, cx * 2 + 2)
                        ty = ((cy * P2I) << 1, ((cy + 1) * P2I) << 1)
                        tz = ((cz * P3I) << 1, ((cz + 1) * P3I) << 1)
                    a0 = a1 = a2 = a3 = None
                    for i in range(8):
                        ox, oy, oz = i & 1, (i >> 1) & 1, (i >> 2) & 1
                        if dense:
                            wi = tx[ox] + ty[oy] + tz[oz]
                        else:
                            wi = (tx[ox] ^ ty[oy] ^ tz[oz]) & 65534
                        w0 = plsc.load_gather(tbl_v, [wi])
                        w1 = plsc.load_gather(tbl_v, [wi + 1])
                        t0 = plsc.bitcast(w0 << 16, jnp.float32)
                        t1 = plsc.bitcast(w0 & M16, jnp.float32)
                        t2 = plsc.bitcast(w1 << 16, jnp.float32)
                        t3 = plsc.bitcast(w1 & M16, jnp.float32)
                        wt = axy[(oy << 1) | ox] * (wz if oz else uz)
                        if a0 is None:
                            a0, a1, a2, a3 = t0 * wt, t1 * wt, t2 * wt, t3 * wt
                        else:
                            a0 += t0 * wt
                            a1 += t1 * wt
                            a2 += t2 * wt
                            a3 += t3 * wt
                    f0v[sl] = a0
                    f1v[sl] = a1
                    f2v[sl] = a2
                    f3v[sl] = a3

                pltpu.sync_copy(f0v, outs[4 * lvl + 0].at[sl_hbm])
                pltpu.sync_copy(f1v, outs[4 * lvl + 1].at[sl_hbm])
                pltpu.sync_copy(f2v, outs[4 * lvl + 2].at[sl_hbm])
                pltpu.sync_copy(f3v, outs[4 * lvl + 3].at[sl_hbm])
                return 0

            lax.fori_loop(0, npts // CP, chunk_body, 0)


def _make_sc_encode(n):
    return pl.kernel(
        _sc_encode_body,
        out_type=[jax.ShapeDtypeStruct((n,), jnp.float32) for _ in range(16)],
        mesh=plsc.VectorSubcoreMesh(core_axis_name="c", subcore_axis_name="s"),
        compiler_params=pltpu.CompilerParams(needs_layout_passes=False),
        scratch_types=[
            pltpu.VMEM((CP,), jnp.float32),
            pltpu.VMEM((CP,), jnp.float32),
            pltpu.VMEM((CP,), jnp.float32),
            pltpu.VMEM((2 * T,), jnp.int32),
            pltpu.VMEM((CP,), jnp.float32),
            pltpu.VMEM((CP,), jnp.float32),
            pltpu.VMEM((CP,), jnp.float32),
            pltpu.VMEM((CP,), jnp.float32),
        ],
    )


def _mlp_body(*refs):
    feats = refs[:16]
    w1, b1, w2, b2, w3, b3, w4, b4, out_ref = refs[16:]
    x = jnp.concatenate([f[...].reshape(1, BN) for f in feats], axis=0)
    h = jnp.dot(w1[...], x, preferred_element_type=jnp.float32) + b1[...]
    h = jnp.where(h >= 0, h, 0.2 * h)
    h = jnp.dot(w2[...], h, preferred_element_type=jnp.float32) + b2[...]
    h = jnp.where(h >= 0, h, 0.2 * h)
    h = jnp.dot(w3[...], h, preferred_element_type=jnp.float32) + b3[...]
    h = jnp.where(h >= 0, h, 0.2 * h)
    o = jnp.dot(w4[...], h, preferred_element_type=jnp.float32) + b4[...]
    out_ref[...] = jnp.tanh(o)


def _mlp_call(n):
    full = lambda shape: pl.BlockSpec(shape, lambda i: tuple(0 for _ in shape))
    lane = pl.BlockSpec((BN,), lambda i: (i,))
    return pl.pallas_call(
        _mlp_body,
        grid=(n // BN,),
        in_specs=[lane] * 16 + [
            full((32, 16)), full((32, 1)),
            full((16, 32)), full((16, 1)),
            full((8, 16)), full((8, 1)),
            full((3, 8)), full((3, 1)),
        ],
        out_specs=pl.BlockSpec((3, BN), lambda i: (0, i)),
        out_shape=jax.ShapeDtypeStruct((3, n), jnp.float32),
    )


def _pack_tables(tables):
    tb = tables.astype(jnp.bfloat16)
    tu = lax.bitcast_convert_type(tb, jnp.uint16).astype(jnp.uint32)
    w0 = tu[..., 0] | (tu[..., 1] << 16)
    w1 = tu[..., 2] | (tu[..., 3] << 16)
    packed = jnp.stack([w0, w1], axis=-1).reshape(L * 2 * T)
    return lax.bitcast_convert_type(packed, jnp.int32)


def kernel(directions, tables, W1, b1, W2, b2, W3, b3, W4, b4):
    n = directions.shape[0]
    assert n % (WPG * CP) == 0 and n % BN == 0
    x = directions[:, 0]
    y = directions[:, 1]
    z = directions[:, 2]
    packed = _pack_tables(tables)
    feats = _make_sc_encode(n)(x, y, z, packed)
    out_t = _mlp_call(n)(
        *feats,
        W1.T, b1.reshape(32, 1),
        W2.T, b2.reshape(16, 1),
        W3.T, b3.reshape(8, 1),
        W4.T, b4.reshape(3, 1),
    )
    return out_t.T


# speedup vs baseline: 1.0429x; 1.0429x over previous
import functools

import numpy as np
import jax
import jax.numpy as jnp
from jax import lax
from jax.experimental import pallas as pl
from jax.experimental.pallas import tpu as pltpu
from jax.experimental.pallas import tpu_sc as plsc

T = 32768
L = 4
RES = [16, 24, 36, 54]
ROWS = [(r + 1) ** 3 if (r + 1) ** 3 <= T else T for r in RES]
P2I = int(np.int32(np.uint32(2654435761)))
P3I = int(np.int32(np.uint32(805459861)))
M16 = -65536
NW_PAD = [-(-2 * r // 512) * 512 for r in ROWS]

NC, NS = 2, 16
NW = NC * NS
WPG = NW // L
CP = 2048

BN = 16384


def _sc_encode_body(x_hbm, y_hbm, z_hbm, tbl_hbm, *rest):
    outs = rest[:16]
    xv_r, yv_r, zv_r, tbl_v, f0v, f1v, f2v, f3v = rest[16:]
    npts_total = x_hbm.shape[0]
    npts = npts_total // WPG
    wid = lax.axis_index("s") * NC + lax.axis_index("c")
    g = wid // WPG
    lw = wid % WPG

    for lvl in range(L):
        @pl.when(g == lvl)
        def _(lvl=lvl):
            res = RES[lvl]
            r1 = res + 1
            dense = r1 ** 3 <= T
            nwords = NW_PAD[lvl]
            pltpu.sync_copy(tbl_hbm.at[pl.ds(lvl * 2 * T, nwords)],
                            tbl_v.at[pl.ds(0, nwords)])
            base = lw * npts

            def chunk_body(c, _):
                cbase = base + c * CP
                sl_hbm = pl.ds(cbase, CP)
                pltpu.sync_copy(x_hbm.at[sl_hbm], xv_r)
                pltpu.sync_copy(y_hbm.at[sl_hbm], yv_r)
                pltpu.sync_copy(z_hbm.at[sl_hbm], zv_r)

                @plsc.parallel_loop(0, CP // 16, 1, unroll=2)
                def vreg_body(j):
                    sl = pl.ds(j * 16, 16)
                    x = xv_r[sl]
                    y = yv_r[sl]
                    z = zv_r[sl]
                    fres = float(res)
                    px = x * fres
                    py = y * fres
                    pz = z * fres
                    cx = px.astype(jnp.int32)
                    cy = py.astype(jnp.int32)
                    cz = pz.astype(jnp.int32)
                    wx = px - cx.astype(jnp.float32)
                    wy = py - cy.astype(jnp.float32)
                    wz = pz - cz.astype(jnp.float32)
                    ux = 1.0 - wx
                    uy = 1.0 - wy
                    uz = 1.0 - wz
                    axy = (ux * uy, wx * uy, ux * wy, wx * wy)
                    if dense:
                        tx = (cx * 2, cx * 2 + 2)
                        ty = (cy * (2 * r1), (cy + 1) * (2 * r1))
                        tz = (cz * (2 * r1 * r1), (cz + 1) * (2 * r1 * r1))
                    else:
                        tx = (cx * 2, cx * 2 + 2)
                        ty = ((cy * P2I) << 1, ((cy + 1) * P2I) << 1)
                        tz = ((cz * P3I) << 1, ((cz + 1) * P3I) << 1)
                    a0 = a1 = a2 = a3 = None
                    for i in range(8):
                        ox, oy, oz = i & 1, (i >> 1) & 1, (i >> 2) & 1
                        if dense:
                            wi = tx[ox] + ty[oy] + tz[oz]
                        else:
                            wi = (tx[ox] ^ ty[oy] ^ tz[oz]) & 65534
                        w0 = plsc.load_gather(tbl_v, [wi])
                        w1 = plsc.load_gather(tbl_v, [wi + 1])
                        t0 = plsc.bitcast(w0 << 16, jnp.float32)
                        t1 = plsc.bitcast(w0 & M16, jnp.float32)
                        t2 = plsc.bitcast(w1 << 16, jnp.float32)
                        t3 = plsc.bitcast(w1 & M16, jnp.float32)
                        wt = axy[(oy << 1) | ox] * (wz if oz else uz)
                        if a0 is None:
                            a0, a1, a2, a3 = t0 * wt, t1 * wt, t2 * wt, t3 * wt
                        else:
                            a0 += t0 * wt
                            a1 += t1 * wt
                            a2 += t2 * wt
                            a3 += t3 * wt
                    f0v[sl] = a0
                    f1v[sl] = a1
                    f2v[sl] = a2
                    f3v[sl] = a3

                pltpu.sync_copy(f0v, outs[4 * lvl + 0].at[sl_hbm])
                pltpu.sync_copy(f1v, outs[4 * lvl + 1].at[sl_hbm])
                pltpu.sync_copy(f2v, outs[4 * lvl + 2].at[sl_hbm])
                pltpu.sync_copy(f3v, outs[4 * lvl + 3].at[sl_hbm])
                return 0

            lax.fori_loop(0, npts // CP, chunk_body, 0)


def _make_sc_encode(n):
    return pl.kernel(
        _sc_encode_body,
        out_type=[jax.ShapeDtypeStruct((n,), jnp.float32) for _ in range(16)],
        mesh=plsc.VectorSubcoreMesh(core_axis_name="c", subcore_axis_name="s"),
        compiler_params=pltpu.CompilerParams(needs_layout_passes=False),
        scratch_types=[
            pltpu.VMEM((CP,), jnp.float32),
            pltpu.VMEM((CP,), jnp.float32),
            pltpu.VMEM((CP,), jnp.float32),
            pltpu.VMEM((2 * T,), jnp.int32),
            pltpu.VMEM((CP,), jnp.float32),
            pltpu.VMEM((CP,), jnp.float32),
            pltpu.VMEM((CP,), jnp.float32),
            pltpu.VMEM((CP,), jnp.float32),
        ],
    )


def _mlp_body(*refs):
    feats = refs[:16]
    w1, b1, w2, b2, w3, b3, w4, b4, out_ref = refs[16:]
    x = jnp.concatenate([f[...].reshape(1, BN) for f in feats], axis=0)
    h = jnp.dot(w1[...], x, preferred_element_type=jnp.float32) + b1[...]
    h = jnp.where(h >= 0, h, 0.2 * h)
    h = jnp.dot(w2[...], h, preferred_element_type=jnp.float32) + b2[...]
    h = jnp.where(h >= 0, h, 0.2 * h)
    h = jnp.dot(w3[...], h, preferred_element_type=jnp.float32) + b3[...]
    h = jnp.where(h >= 0, h, 0.2 * h)
    o = jnp.dot(w4[...], h, preferred_element_type=jnp.float32) + b4[...]
    out_ref[...] = jnp.tanh(o)


def _mlp_call(n):
    full = lambda shape: pl.BlockSpec(shape, lambda i: tuple(0 for _ in shape))
    lane = pl.BlockSpec((BN,), lambda i: (i,))
    return pl.pallas_call(
        _mlp_body,
        grid=(n // BN,),
        in_specs=[lane] * 16 + [
            full((32, 16)), full((32, 1)),
            full((16, 32)), full((16, 1)),
            full((8, 16)), full((8, 1)),
            full((3, 8)), full((3, 1)),
        ],
        out_specs=pl.BlockSpec((3, BN), lambda i: (0, i)),
        out_shape=jax.ShapeDtypeStruct((3, n), jnp.float32),
    )


def _pack_tables(tables):
    tb = tables.astype(jnp.bfloat16)
    tu = lax.bitcast_convert_type(tb, jnp.uint16).astype(jnp.uint32)
    w0 = tu[..., 0] | (tu[..., 1] << 16)
    w1 = tu[..., 2] | (tu[..., 3] << 16)
    packed = jnp.stack([w0, w1], axis=-1).reshape(L * 2 * T)
    return lax.bitcast_convert_type(packed, jnp.int32)


def kernel(directions, tables, W1, b1, W2, b2, W3, b3, W4, b4):
    n = directions.shape[0]
    assert n % (WPG * CP) == 0 and n % BN == 0
    x = directions[:, 0]
    y = directions[:, 1]
    z = directions[:, 2]
    packed = _pack_tables(tables)
    feats = _make_sc_encode(n)(x, y, z, packed)
    out_t = _mlp_call(n)(
        *feats,
        W1.T, b1.reshape(32, 1),
        W2.T, b2.reshape(16, 1),
        W3.T, b3.reshape(8, 1),
        W4.T, b4.reshape(3, 1),
    )
    return out_t.T


# speedup vs baseline: 1.0698x; 1.0258x over previous
import functools

import numpy as np
import jax
import jax.numpy as jnp
from jax import lax
from jax.experimental import pallas as pl
from jax.experimental.pallas import tpu as pltpu
from jax.experimental.pallas import tpu_sc as plsc

T = 32768
L = 4
RES = [16, 24, 36, 54]
ROWS = [(r + 1) ** 3 if (r + 1) ** 3 <= T else T for r in RES]
P2I = int(np.int32(np.uint32(2654435761)))
P3I = int(np.int32(np.uint32(805459861)))
M16 = -65536
NW_PAD = [-(-2 * r // 512) * 512 for r in ROWS]

NC, NS = 2, 16
NW = NC * NS
WPG = NW // L
CP = 2048

BN = 16384


def _sc_encode_body(x_hbm, y_hbm, z_hbm, tbl_hbm, *rest):
    outs = rest[:16]
    xv_r, yv_r, zv_r, tbl_v, f0v, f1v, f2v, f3v = rest[16:]
    npts_total = x_hbm.shape[0]
    npts = npts_total // WPG
    wid = lax.axis_index("s") * NC + lax.axis_index("c")
    g = wid // WPG
    lw = wid % WPG

    for lvl in range(L):
        @pl.when(g == lvl)
        def _(lvl=lvl):
            res = RES[lvl]
            r1 = res + 1
            dense = r1 ** 3 <= T
            nwords = NW_PAD[lvl]
            pltpu.sync_copy(tbl_hbm.at[pl.ds(lvl * 2 * T, nwords)],
                            tbl_v.at[pl.ds(0, nwords)])
            base = lw * npts

            def chunk_body(c, _):
                cbase = base + c * CP
                sl_hbm = pl.ds(cbase, CP)
                pltpu.sync_copy(x_hbm.at[sl_hbm], xv_r)
                pltpu.sync_copy(y_hbm.at[sl_hbm], yv_r)
                pltpu.sync_copy(z_hbm.at[sl_hbm], zv_r)

                @plsc.parallel_loop(0, CP // 16, 1, unroll=2)
                def vreg_body(j):
                    sl = pl.ds(j * 16, 16)
                    x = xv_r[sl]
                    y = yv_r[sl]
                    z = zv_r[sl]
                    fres = float(res)
                    px = x * fres
                    py = y * fres
                    pz = z * fres
                    cx = px.astype(jnp.int32)
                    cy = py.astype(jnp.int32)
                    cz = pz.astype(jnp.int32)
                    wx = px - cx.astype(jnp.float32)
                    wy = py - cy.astype(jnp.float32)
                    wz = pz - cz.astype(jnp.float32)
                    ux = 1.0 - wx
                    uy = 1.0 - wy
                    uz = 1.0 - wz
                    axy = (ux * uy, wx * uy, ux * wy, wx * wy)
                    if dense:
                        tx = (cx * 2, cx * 2 + 2)
                        ty = (cy * (2 * r1), (cy + 1) * (2 * r1))
                        tz = (cz * (2 * r1 * r1), (cz + 1) * (2 * r1 * r1))
                    else:
                        tx = (cx * 2, cx * 2 + 2)
                        ty = ((cy * P2I) << 1, ((cy + 1) * P2I) << 1)
                        tz = ((cz * P3I) << 1, ((cz + 1) * P3I) << 1)
                    a0 = a1 = a2 = a3 = None
                    for i in range(8):
                        ox, oy, oz = i & 1, (i >> 1) & 1, (i >> 2) & 1
                        if dense:
                            wi = tx[ox] + ty[oy] + tz[oz]
                        else:
                            wi = (tx[ox] ^ ty[oy] ^ tz[oz]) & 65534
                        w0 = plsc.load_gather(tbl_v, [wi])
                        w1 = plsc.load_gather(tbl_v, [wi + 1])
                        t0 = plsc.bitcast(w0 << 16, jnp.float32)
                        t1 = plsc.bitcast(w0 & M16, jnp.float32)
                        t2 = plsc.bitcast(w1 << 16, jnp.float32)
                        t3 = plsc.bitcast(w1 & M16, jnp.float32)
                        wt = axy[(oy << 1) | ox] * (wz if oz else uz)
                        if a0 is None:
                            a0, a1, a2, a3 = t0 * wt, t1 * wt, t2 * wt, t3 * wt
                        else:
                            a0 += t0 * wt
                            a1 += t1 * wt
                            a2 += t2 * wt
                            a3 += t3 * wt
                    f0v[sl] = a0
                    f1v[sl] = a1
                    f2v[sl] = a2
                    f3v[sl] = a3

                pltpu.sync_copy(f0v, outs[4 * lvl + 0].at[sl_hbm])
                pltpu.sync_copy(f1v, outs[4 * lvl + 1].at[sl_hbm])
                pltpu.sync_copy(f2v, outs[4 * lvl + 2].at[sl_hbm])
                pltpu.sync_copy(f3v, outs[4 * lvl + 3].at[sl_hbm])
                return 0

            lax.fori_loop(0, npts // CP, chunk_body, 0)


def _make_sc_encode(n):
    return pl.kernel(
        _sc_encode_body,
        out_type=[jax.ShapeDtypeStruct((n,), jnp.float32) for _ in range(16)],
        mesh=plsc.VectorSubcoreMesh(core_axis_name="c", subcore_axis_name="s"),
        compiler_params=pltpu.CompilerParams(needs_layout_passes=False),
        scratch_types=[
            pltpu.VMEM((CP,), jnp.float32),
            pltpu.VMEM((CP,), jnp.float32),
            pltpu.VMEM((CP,), jnp.float32),
            pltpu.VMEM((2 * T,), jnp.int32),
            pltpu.VMEM((CP,), jnp.float32),
            pltpu.VMEM((CP,), jnp.float32),
            pltpu.VMEM((CP,), jnp.float32),
            pltpu.VMEM((CP,), jnp.float32),
        ],
    )


def _mlp_body(*refs):
    feats = refs[:16]
    w1, b1, w2, b2, w3, b3, w4, b4, out_ref = refs[16:]
    x = jnp.concatenate([f[...].reshape(1, BN) for f in feats], axis=0)
    h = jnp.dot(w1[...], x, preferred_element_type=jnp.float32) + b1[...]
    h = jnp.where(h >= 0, h, 0.2 * h)
    h = jnp.dot(w2[...], h, preferred_element_type=jnp.float32) + b2[...]
    h = jnp.where(h >= 0, h, 0.2 * h)
    h = jnp.dot(w3[...], h, preferred_element_type=jnp.float32) + b3[...]
    h = jnp.where(h >= 0, h, 0.2 * h)
    o = jnp.dot(w4[...], h, preferred_element_type=jnp.float32) + b4[...]
    out_ref[...] = jnp.tanh(o)


def _mlp_call(n):
    full = lambda shape: pl.BlockSpec(shape, lambda i: tuple(0 for _ in shape))
    lane = pl.BlockSpec((BN,), lambda i: (i,))
    return pl.pallas_call(
        _mlp_body,
        grid=(n // BN,),
        in_specs=[lane] * 16 + [
            full((32, 16)), full((32, 1)),
            full((16, 32)), full((16, 1)),
            full((8, 16)), full((8, 1)),
            full((3, 8)), full((3, 1)),
        ],
        out_specs=pl.BlockSpec((3, BN), lambda i: (0, i)),
        out_shape=jax.ShapeDtypeStruct((3, n), jnp.float32),
    )


def _pack_tables(tables):
    tb = tables.astype(jnp.bfloat16)
    tu = lax.bitcast_convert_type(tb, jnp.uint16).astype(jnp.uint32)
    w0 = tu[..., 0] | (tu[..., 1] << 16)
    w1 = tu[..., 2] | (tu[..., 3] << 16)
    packed = jnp.stack([w0, w1], axis=-1).reshape(L * 2 * T)
    return lax.bitcast_convert_type(packed, jnp.int32)


NSEG = 2


def kernel(directions, tables, W1, b1, W2, b2, W3, b3, W4, b4):
    n = directions.shape[0]
    ns = n // NSEG
    assert ns % (WPG * CP) == 0 and ns % BN == 0
    packed = _pack_tables(tables)
    weights = (
        W1.T, b1.reshape(32, 1),
        W2.T, b2.reshape(16, 1),
        W3.T, b3.reshape(8, 1),
        W4.T, b4.reshape(3, 1),
    )
    sc = _make_sc_encode(ns)
    mlp = _mlp_call(ns)
    outs = []
    for s in range(NSEG):
        sl = slice(s * ns, (s + 1) * ns)
        feats = sc(directions[sl, 0], directions[sl, 1], directions[sl, 2],
                   packed)
        outs.append(mlp(*feats, *weights))
    return jnp.concatenate([o.T for o in outs], axis=0)


# speedup vs baseline: 1.3958x; 1.3047x over previous
import functools

import numpy as np
import jax
import jax.numpy as jnp
from jax import lax
from jax.experimental import pallas as pl
from jax.experimental.pallas import tpu as pltpu
from jax.experimental.pallas import tpu_sc as plsc

T = 32768
L = 4
RES = [16, 24, 36, 54]
ROWS = [(r + 1) ** 3 if (r + 1) ** 3 <= T else T for r in RES]
P2I = int(np.int32(np.uint32(2654435761)))
P3I = int(np.int32(np.uint32(805459861)))
M16 = -65536
NW_PAD = [-(-2 * r // 512) * 512 for r in ROWS]

NC, NS = 2, 16
NW = NC * NS
WPG = NW // L
CP = 2048

BN = 16384


def _sc_encode_body(x_hbm, y_hbm, z_hbm, tbl_hbm, *rest):
    outs = rest[:16]
    tbl_v = rest[16]
    xv = rest[17:19]
    yv = rest[19:21]
    zv = rest[21:23]
    fb = (rest[23:27], rest[27:31])
    isem = rest[31:33]
    osem = rest[33:35]
    npts_total = x_hbm.shape[0]
    npts = npts_total // WPG
    nch = npts // CP
    wid = lax.axis_index("s") * NC + lax.axis_index("c")
    g = wid // WPG
    lw = wid % WPG

    for lvl in range(L):
        @pl.when(g == lvl)
        def _(lvl=lvl):
            res = RES[lvl]
            r1 = res + 1
            dense = r1 ** 3 <= T
            nwords = NW_PAD[lvl]
            pltpu.sync_copy(tbl_hbm.at[pl.ds(lvl * 2 * T, nwords)],
                            tbl_v.at[pl.ds(0, nwords)])
            base = lw * npts

            def in_start(c, b):
                sl = pl.ds(base + c * CP, CP)
                pltpu.async_copy(x_hbm.at[sl], xv[b], isem[b])
                pltpu.async_copy(y_hbm.at[sl], yv[b], isem[b])
                pltpu.async_copy(z_hbm.at[sl], zv[b], isem[b])

            def in_wait(b):
                sl0 = pl.ds(0, CP)
                pltpu.make_async_copy(x_hbm.at[sl0], xv[b], isem[b]).wait()
                pltpu.make_async_copy(y_hbm.at[sl0], yv[b], isem[b]).wait()
                pltpu.make_async_copy(z_hbm.at[sl0], zv[b], isem[b]).wait()

            def out_start(c, b):
                sl = pl.ds(base + c * CP, CP)
                for k in range(4):
                    pltpu.async_copy(fb[b][k], outs[4 * lvl + k].at[sl], osem[b])

            def out_wait(b):
                sl0 = pl.ds(0, CP)
                for k in range(4):
                    pltpu.make_async_copy(fb[b][k], outs[4 * lvl + k].at[sl0],
                                          osem[b]).wait()

            def compute(b):
                xv_r, yv_r, zv_r = xv[b], yv[b], zv[b]
                f0v, f1v, f2v, f3v = fb[b]

                @plsc.parallel_loop(0, CP // 16, 1, unroll=2)
                def vreg_body(j):
                    sl = pl.ds(j * 16, 16)
                    x = xv_r[sl]
                    y = yv_r[sl]
                    z = zv_r[sl]
                    fres = float(res)
                    px = x * fres
                    py = y * fres
                    pz = z * fres
                    cx = px.astype(jnp.int32)
                    cy = py.astype(jnp.int32)
                    cz = pz.astype(jnp.int32)
                    wx = px - cx.astype(jnp.float32)
                    wy = py - cy.astype(jnp.float32)
                    wz = pz - cz.astype(jnp.float32)
                    ux = 1.0 - wx
                    uy = 1.0 - wy
                    uz = 1.0 - wz
                    axy = (ux * uy, wx * uy, ux * wy, wx * wy)
                    if dense:
                        tx = (cx * 2, cx * 2 + 2)
                        ty = (cy * (2 * r1), (cy + 1) * (2 * r1))
                        tz = (cz * (2 * r1 * r1), (cz + 1) * (2 * r1 * r1))
                    else:
                        tx = (cx * 2, cx * 2 + 2)
                        ty = ((cy * P2I) << 1, ((cy + 1) * P2I) << 1)
                        tz = ((cz * P3I) << 1, ((cz + 1) * P3I) << 1)
                    a0 = a1 = a2 = a3 = None
                    for i in range(8):
                        ox, oy, oz = i & 1, (i >> 1) & 1, (i >> 2) & 1
                        if dense:
                            wi = tx[ox] + ty[oy] + tz[oz]
                        else:
                            wi = (tx[ox] ^ ty[oy] ^ tz[oz]) & 65534
                        w0 = plsc.load_gather(tbl_v, [wi])
                        w1 = plsc.load_gather(tbl_v, [wi + 1])
                        t0 = plsc.bitcast(w0 << 16, jnp.float32)
                        t1 = plsc.bitcast(w0 & M16, jnp.float32)
                        t2 = plsc.bitcast(w1 << 16, jnp.float32)
                        t3 = plsc.bitcast(w1 & M16, jnp.float32)
                        wt = axy[(oy << 1) | ox] * (wz if oz else uz)
                        if a0 is None:
                            a0, a1, a2, a3 = t0 * wt, t1 * wt, t2 * wt, t3 * wt
                        else:
                            a0 += t0 * wt
                            a1 += t1 * wt
                            a2 += t2 * wt
                            a3 += t3 * wt
                    f0v[sl] = a0
                    f1v[sl] = a1
                    f2v[sl] = a2
                    f3v[sl] = a3

            in_start(0, 0)

            def body2(t, _):
                c0 = t * 2
                for b in (0, 1):
                    c = c0 + b

                    @pl.when(c < nch - 1)
                    def _(c=c, b=b):
                        in_start(c + 1, b ^ 1)

                    in_wait(b)

                    @pl.when(c >= 2)
                    def _(b=b):
                        out_wait(b)

                    compute(b)
                    out_start(c, b)
                return 0

            lax.fori_loop(0, nch // 2, body2, 0)
            out_wait(0)
            out_wait(1)


def _make_sc_encode(n):
    return pl.kernel(
        _sc_encode_body,
        out_type=[jax.ShapeDtypeStruct((n,), jnp.float32) for _ in range(16)],
        mesh=plsc.VectorSubcoreMesh(core_axis_name="c", subcore_axis_name="s"),
        compiler_params=pltpu.CompilerParams(needs_layout_passes=False),
        scratch_types=(
            [pltpu.VMEM((2 * T,), jnp.int32)]
            + [pltpu.VMEM((CP,), jnp.float32)] * 6
            + [pltpu.VMEM((CP,), jnp.float32)] * 8
            + [pltpu.SemaphoreType.DMA] * 4
        ),
    )


def _mlp_body(*refs):
    feats = refs[:16]
    w1, b1, w2, b2, w3, b3, w4, b4, out_ref = refs[16:]
    x = jnp.concatenate([f[...].reshape(1, BN) for f in feats], axis=0)
    h = jnp.dot(w1[...], x, preferred_element_type=jnp.float32) + b1[...]
    h = jnp.where(h >= 0, h, 0.2 * h)
    h = jnp.dot(w2[...], h, preferred_element_type=jnp.float32) + b2[...]
    h = jnp.where(h >= 0, h, 0.2 * h)
    h = jnp.dot(w3[...], h, preferred_element_type=jnp.float32) + b3[...]
    h = jnp.where(h >= 0, h, 0.2 * h)
    o = jnp.dot(w4[...], h, preferred_element_type=jnp.float32) + b4[...]
    out_ref[...] = jnp.tanh(o)


def _mlp_call(n):
    full = lambda shape: pl.BlockSpec(shape, lambda i: tuple(0 for _ in shape))
    lane = pl.BlockSpec((BN,), lambda i: (i,))
    return pl.pallas_call(
        _mlp_body,
        grid=(n // BN,),
        in_specs=[lane] * 16 + [
            full((32, 16)), full((32, 1)),
            full((16, 32)), full((16, 1)),
            full((8, 16)), full((8, 1)),
            full((3, 8)), full((3, 1)),
        ],
        out_specs=pl.BlockSpec((3, BN), lambda i: (0, i)),
        out_shape=jax.ShapeDtypeStruct((3, n), jnp.float32),
    )


def _pack_tables(tables):
    tb = tables.astype(jnp.bfloat16)
    tu = lax.bitcast_convert_type(tb, jnp.uint16).astype(jnp.uint32)
    w0 = tu[..., 0] | (tu[..., 1] << 16)
    w1 = tu[..., 2] | (tu[..., 3] << 16)
    packed = jnp.stack([w0, w1], axis=-1).reshape(L * 2 * T)
    return lax.bitcast_convert_type(packed, jnp.int32)


NSEG = 2


def kernel(directions, tables, W1, b1, W2, b2, W3, b3, W4, b4):
    n = directions.shape[0]
    ns = n // NSEG
    assert ns % (WPG * CP) == 0 and ns % BN == 0
    packed = _pack_tables(tables)
    weights = (
        W1.T, b1.reshape(32, 1),
        W2.T, b2.reshape(16, 1),
        W3.T, b3.reshape(8, 1),
        W4.T, b4.reshape(3, 1),
    )
    sc = _make_sc_encode(ns)
    mlp = _mlp_call(ns)
    outs = []
    for s in range(NSEG):
        sl = slice(s * ns, (s + 1) * ns)
        feats = sc(directions[sl, 0], directions[sl, 1], directions[sl, 2],
                   packed)
        outs.append(mlp(*feats, *weights))
    return jnp.concatenate([o.T for o in outs], axis=0)
